# baseline (speedup 1.0000x reference)
import functools

import jax
import jax.numpy as jnp
from jax import lax
from jax.experimental import pallas as pl
from jax.experimental.pallas import tpu as pltpu
from jax.experimental.pallas import tpu_sc as plsc

T_STEPS = 16
TAU_LIF = 2.0
V_TH = 1.0
BN_EPS = 1e-5


def _bf16_rtne(v):
    u = lax.bitcast_convert_type(v, jnp.int32)
    r = u + ((u >> 16) & 1) + 0x7FFF
    r = r & jnp.int32(-65536)
    return lax.bitcast_convert_type(r, jnp.float32).astype(jnp.bfloat16)


def _memout_body(coef_s, alpha_s, x_ref, xm_ref):
    a = alpha_s[0, 0]
    x0 = x_ref[0, 0]
    s1 = x0 * coef_s[0, 0]
    s2 = x0
    for t in range(1, T_STEPS):
        xt = x_ref[t, 0]
        s1 = s1 + xt * coef_s[0, t]
        s2 = s2 + xt
    xm_ref[0] = (1.0 - a) * s1 + (a * s2) * 0.0625


def _head_body(q_ref, w_ref, cp_ref, out_ref):
    c = w_ref.shape[1]
    q = q_ref[0][:, :c]
    w = w_ref[...]
    h = lax.dot_general(_bf16_rtne(w), _bf16_rtne(q),
                        (((1,), (1,)), ((), ())),
                        preferred_element_type=jnp.float32)
    h = h + cp_ref[:, 0:1]
    h = (h - cp_ref[:, 3:4]) / jnp.sqrt(cp_ref[:, 4:5] + BN_EPS)
    h = h * cp_ref[:, 1:2] + cp_ref[:, 2:3]
    v = jnp.zeros_like(h)
    for t in range(T_STEPS):
        v = v + (h - v) / TAU_LIF
        s = (v - V_TH >= 0.0).astype(jnp.float32)
        out_ref[t, 0] = s
        v = (1.0 - s) * v


def _sc_gather(embeddings, indices):
    n = indices.shape[0]
    k_rows, d = embeddings.shape
    d_pad = 128
    emb_pad = jnp.pad(embeddings, ((0, 0), (0, d_pad - d)))
    info = plsc.get_sparse_core_info()
    nc, ns = info.num_cores, info.num_subcores
    nw = nc * ns
    b_per_w = n // nw
    mesh = plsc.VectorSubcoreMesh(core_axis_name="c", subcore_axis_name="s")

    @functools.partial(
        pl.kernel,
        out_type=jax.ShapeDtypeStruct((n, d), jnp.float32),
        mesh=mesh,
        scratch_types=[
            pltpu.VMEM((b_per_w,), jnp.int32),
            pltpu.VMEM((b_per_w, d_pad), jnp.float32),
            pltpu.VMEM((b_per_w, d), jnp.float32),
            pltpu.SemaphoreType.DMA,
        ],
    )
    def gather_kernel(emb_hbm, idx_hbm, out_hbm, idx_v, rows_v, cmp_v, sem):
        wid = lax.axis_index("s") * nc + lax.axis_index("c")
        base = wid * b_per_w
        pltpu.sync_copy(idx_hbm.at[pl.ds(base, b_per_w)], idx_v)
        pltpu.async_copy(emb_hbm.at[idx_v], rows_v, sem).wait()

        def body(i, _):
            cmp_v[i, :] = rows_v[i, pl.ds(0, d)]
            return 0

        lax.fori_loop(0, b_per_w, body, 0)
        pltpu.sync_copy(cmp_v, out_hbm.at[pl.ds(base, b_per_w)])

    return gather_kernel(emb_pad, indices)


def kernel(x, embeddings, alpha, conv_w, conv_b, bn_gamma, bn_beta,
           bn_mean, bn_var):
    t_steps, b, c, h, w = x.shape
    hw = h * w
    k_rows, d = embeddings.shape
    x4 = x.reshape(t_steps, b, c, hw)
    coef = jnp.power(0.8, jnp.arange(t_steps - 1, -1, -1,
                                     dtype=jnp.float32)).reshape(1, t_steps)
    alpha2 = jnp.asarray(alpha, jnp.float32).reshape(1, 1)

    xm3 = pl.pallas_call(
        _memout_body,
        grid=(b,),
        in_specs=[
            pl.BlockSpec(memory_space=pltpu.SMEM),
            pl.BlockSpec(memory_space=pltpu.SMEM),
            pl.BlockSpec((t_steps, 1, c, hw), lambda bb: (0, bb, 0, 0)),
        ],
        out_specs=pl.BlockSpec((1, c, hw), lambda bb: (bb, 0, 0)),
        out_shape=jax.ShapeDtypeStruct((b, c, hw), jnp.float32),
    )(coef, alpha2, x4)

    flat_x = jnp.transpose(xm3.reshape(b, c, h, w), (0, 2, 3, 1)).reshape(-1, d)
    distances = (jnp.sum(flat_x ** 2, axis=1, keepdims=True)
                 + jnp.sum(embeddings ** 2, axis=1)
                 - 2.0 * jnp.matmul(flat_x, embeddings.T))
    encoding_indices = jnp.argmin(distances, axis=1)

    quantized = _sc_gather(embeddings, encoding_indices)

    cp = jnp.stack([conv_b, bn_gamma, bn_beta, bn_mean, bn_var],
                   axis=1).astype(jnp.float32)
    spikes4 = pl.pallas_call(
        _head_body,
        grid=(b,),
        in_specs=[
            pl.BlockSpec((1, hw, d), lambda bb: (bb, 0, 0)),
            pl.BlockSpec((c, c), lambda bb: (0, 0)),
            pl.BlockSpec(cp.shape, lambda bb: (0, 0)),
        ],
        out_specs=pl.BlockSpec((t_steps, 1, c, hw), lambda bb: (0, bb, 0, 0)),
        out_shape=jax.ShapeDtypeStruct((t_steps, b, c, hw), jnp.float32),
    )(quantized.reshape(b, hw, d), conv_w[:, :, 0, 0], cp)

    return spikes4.reshape(t_steps, b, c, h, w), encoding_indices


# speedup vs baseline: 1.0052x; 1.0052x over previous
import functools

import jax
import jax.numpy as jnp
from jax import lax
from jax.experimental import pallas as pl
from jax.experimental.pallas import tpu as pltpu
from jax.experimental.pallas import tpu_sc as plsc

T_STEPS = 16
TAU_LIF = 2.0
V_TH = 1.0
BN_EPS = 1e-5


def _bf16_rtne(v):
    u = lax.bitcast_convert_type(v, jnp.int32)
    r = u + ((u >> 16) & 1) + 0x7FFF
    r = r & jnp.int32(-65536)
    return lax.bitcast_convert_type(r, jnp.float32).astype(jnp.bfloat16)


def _memout_body(coef_s, alpha_s, x_ref, xm_ref):
    a = alpha_s[0, 0]
    x0 = x_ref[0, 0]
    s1 = x0 * coef_s[0, 0]
    s2 = x0
    for t in range(1, T_STEPS):
        xt = x_ref[t, 0]
        s1 = s1 + xt * coef_s[0, t]
        s2 = s2 + xt
    xm_ref[0] = (1.0 - a) * s1 + (a * s2) * 0.0625


def _head_body(q_ref, w_ref, cp_ref, out_ref):
    c = w_ref.shape[1]
    q = q_ref[0][:, :c]
    w = w_ref[...]
    h = lax.dot_general(_bf16_rtne(w), _bf16_rtne(q),
                        (((1,), (1,)), ((), ())),
                        preferred_element_type=jnp.float32)
    h = h + cp_ref[:, 0:1]
    h = (h - cp_ref[:, 3:4]) / jnp.sqrt(cp_ref[:, 4:5] + BN_EPS)
    h = h * cp_ref[:, 1:2] + cp_ref[:, 2:3]
    v = jnp.zeros_like(h)
    for t in range(T_STEPS):
        v = v + (h - v) / TAU_LIF
        s = (v - V_TH >= 0.0).astype(jnp.float32)
        out_ref[t, 0] = s
        v = (1.0 - s) * v


def _sc_gather(embeddings, indices):
    n = indices.shape[0]
    k_rows, d = embeddings.shape
    d_pad = 128
    emb_pad = jnp.pad(embeddings, ((0, 0), (0, d_pad - d)))
    info = plsc.get_sparse_core_info()
    nc, ns = info.num_cores, info.num_subcores
    nw = nc * ns
    b_per_w = n // nw
    mesh = plsc.VectorSubcoreMesh(core_axis_name="c", subcore_axis_name="s")

    @functools.partial(
        pl.kernel,
        out_type=jax.ShapeDtypeStruct((n, d_pad), jnp.float32),
        mesh=mesh,
        scratch_types=[
            pltpu.VMEM((b_per_w,), jnp.int32),
            pltpu.VMEM((b_per_w, d_pad), jnp.float32),
            pltpu.SemaphoreType.DMA,
        ],
    )
    def gather_kernel(emb_hbm, idx_hbm, out_hbm, idx_v, rows_v, sem):
        wid = lax.axis_index("s") * nc + lax.axis_index("c")
        base = wid * b_per_w
        pltpu.sync_copy(idx_hbm.at[pl.ds(base, b_per_w)], idx_v)
        pltpu.async_copy(emb_hbm.at[idx_v], rows_v, sem).wait()
        pltpu.sync_copy(rows_v, out_hbm.at[pl.ds(base, b_per_w)])

    return gather_kernel(emb_pad, indices)


def kernel(x, embeddings, alpha, conv_w, conv_b, bn_gamma, bn_beta,
           bn_mean, bn_var):
    t_steps, b, c, h, w = x.shape
    hw = h * w
    k_rows, d = embeddings.shape
    x4 = x.reshape(t_steps, b, c, hw)
    coef = jnp.power(0.8, jnp.arange(t_steps - 1, -1, -1,
                                     dtype=jnp.float32)).reshape(1, t_steps)
    alpha2 = jnp.asarray(alpha, jnp.float32).reshape(1, 1)

    xm3 = pl.pallas_call(
        _memout_body,
        grid=(b,),
        in_specs=[
            pl.BlockSpec(memory_space=pltpu.SMEM),
            pl.BlockSpec(memory_space=pltpu.SMEM),
            pl.BlockSpec((t_steps, 1, c, hw), lambda bb: (0, bb, 0, 0)),
        ],
        out_specs=pl.BlockSpec((1, c, hw), lambda bb: (bb, 0, 0)),
        out_shape=jax.ShapeDtypeStruct((b, c, hw), jnp.float32),
    )(coef, alpha2, x4)

    flat_x = jnp.transpose(xm3.reshape(b, c, h, w), (0, 2, 3, 1)).reshape(-1, d)
    distances = (jnp.sum(flat_x ** 2, axis=1, keepdims=True)
                 + jnp.sum(embeddings ** 2, axis=1)
                 - 2.0 * jnp.matmul(flat_x, embeddings.T))
    encoding_indices = jnp.argmin(distances, axis=1)

    quantized = _sc_gather(embeddings, encoding_indices)
    d_pad = quantized.shape[1]

    cp = jnp.stack([conv_b, bn_gamma, bn_beta, bn_mean, bn_var],
                   axis=1).astype(jnp.float32)
    spikes4 = pl.pallas_call(
        _head_body,
        grid=(b,),
        in_specs=[
            pl.BlockSpec((1, hw, d_pad), lambda bb: (bb, 0, 0)),
            pl.BlockSpec((c, c), lambda bb: (0, 0)),
            pl.BlockSpec(cp.shape, lambda bb: (0, 0)),
        ],
        out_specs=pl.BlockSpec((t_steps, 1, c, hw), lambda bb: (0, bb, 0, 0)),
        out_shape=jax.ShapeDtypeStruct((t_steps, b, c, hw), jnp.float32),
    )(quantized.reshape(b, hw, d_pad), conv_w[:, :, 0, 0], cp)

    return spikes4.reshape(t_steps, b, c, h, w), encoding_indices


# speedup vs baseline: 1.0135x; 1.0083x over previous
import functools

import jax
import jax.numpy as jnp
from jax import lax
from jax.experimental import pallas as pl
from jax.experimental.pallas import tpu as pltpu
from jax.experimental.pallas import tpu_sc as plsc

T_STEPS = 16
TAU_LIF = 2.0
V_TH = 1.0
BN_EPS = 1e-5


def _bf16_rtne(v):
    u = lax.bitcast_convert_type(v, jnp.int32)
    r = u + ((u >> 16) & 1) + 0x7FFF
    r = r & jnp.int32(-65536)
    return lax.bitcast_convert_type(r, jnp.float32).astype(jnp.bfloat16)


def _memout_body(coef_s, alpha_s, x_ref, xm_ref):
    a = alpha_s[0, 0]
    x0 = x_ref[0]
    s1 = x0 * coef_s[0, 0]
    s2 = x0
    for t in range(1, T_STEPS):
        xt = x_ref[t]
        s1 = s1 + xt * coef_s[0, t]
        s2 = s2 + xt
    xm_ref[...] = (1.0 - a) * s1 + (a * s2) * 0.0625


def _head_body(q_ref, w_ref, cp_ref, out_ref):
    c = w_ref.shape[1]
    q = q_ref[0][:, :c]
    w = w_ref[...]
    h = lax.dot_general(_bf16_rtne(w), _bf16_rtne(q),
                        (((1,), (1,)), ((), ())),
                        preferred_element_type=jnp.float32)
    h = h + cp_ref[:, 0:1]
    h = (h - cp_ref[:, 3:4]) / jnp.sqrt(cp_ref[:, 4:5] + BN_EPS)
    h = h * cp_ref[:, 1:2] + cp_ref[:, 2:3]
    v = jnp.zeros_like(h)
    for t in range(T_STEPS):
        v = v + (h - v) / TAU_LIF
        s = (v - V_TH >= 0.0).astype(jnp.float32)
        out_ref[t, 0] = s
        v = (1.0 - s) * v


def _sc_gather(embeddings, indices):
    n = indices.shape[0]
    k_rows, d = embeddings.shape
    d_pad = 128
    emb_pad = jnp.pad(embeddings, ((0, 0), (0, d_pad - d)))
    info = plsc.get_sparse_core_info()
    nc, ns = info.num_cores, info.num_subcores
    nw = nc * ns
    b_per_w = n // nw
    mesh = plsc.VectorSubcoreMesh(core_axis_name="c", subcore_axis_name="s")

    @functools.partial(
        pl.kernel,
        out_type=jax.ShapeDtypeStruct((n, d_pad), jnp.float32),
        mesh=mesh,
        scratch_types=[
            pltpu.VMEM((b_per_w,), jnp.int32),
            pltpu.VMEM((b_per_w, d_pad), jnp.float32),
            pltpu.SemaphoreType.DMA,
        ],
    )
    def gather_kernel(emb_hbm, idx_hbm, out_hbm, idx_v, rows_v, sem):
        wid = lax.axis_index("s") * nc + lax.axis_index("c")
        base = wid * b_per_w
        pltpu.sync_copy(idx_hbm.at[pl.ds(base, b_per_w)], idx_v)
        pltpu.async_copy(emb_hbm.at[idx_v], rows_v, sem).wait()
        pltpu.sync_copy(rows_v, out_hbm.at[pl.ds(base, b_per_w)])

    return gather_kernel(emb_pad, indices)


def kernel(x, embeddings, alpha, conv_w, conv_b, bn_gamma, bn_beta,
           bn_mean, bn_var):
    t_steps, b, c, h, w = x.shape
    hw = h * w
    k_rows, d = embeddings.shape
    x4 = x.reshape(t_steps, b, c, hw)
    coef = jnp.power(0.8, jnp.arange(t_steps - 1, -1, -1,
                                     dtype=jnp.float32)).reshape(1, t_steps)
    alpha2 = jnp.asarray(alpha, jnp.float32).reshape(1, 1)

    bpair = 2
    xm3 = pl.pallas_call(
        _memout_body,
        grid=(b // bpair,),
        in_specs=[
            pl.BlockSpec(memory_space=pltpu.SMEM),
            pl.BlockSpec(memory_space=pltpu.SMEM),
            pl.BlockSpec((t_steps, bpair, c, hw), lambda bb: (0, bb, 0, 0)),
        ],
        out_specs=pl.BlockSpec((bpair, c, hw), lambda bb: (bb, 0, 0)),
        out_shape=jax.ShapeDtypeStruct((b, c, hw), jnp.float32),
    )(coef, alpha2, x4)

    flat_x = jnp.transpose(xm3.reshape(b, c, h, w), (0, 2, 3, 1)).reshape(-1, d)
    distances = (jnp.sum(flat_x ** 2, axis=1, keepdims=True)
                 + jnp.sum(embeddings ** 2, axis=1)
                 - 2.0 * jnp.matmul(flat_x, embeddings.T))
    encoding_indices = jnp.argmin(distances, axis=1)

    quantized = _sc_gather(embeddings, encoding_indices)
    d_pad = quantized.shape[1]

    cp = jnp.stack([conv_b, bn_gamma, bn_beta, bn_mean, bn_var],
                   axis=1).astype(jnp.float32)
    spikes4 = pl.pallas_call(
        _head_body,
        grid=(b,),
        in_specs=[
            pl.BlockSpec((1, hw, d_pad), lambda bb: (bb, 0, 0)),
            pl.BlockSpec((c, c), lambda bb: (0, 0)),
            pl.BlockSpec(cp.shape, lambda bb: (0, 0)),
        ],
        out_specs=pl.BlockSpec((t_steps, 1, c, hw), lambda bb: (0, bb, 0, 0)),
        out_shape=jax.ShapeDtypeStruct((t_steps, b, c, hw), jnp.float32),
    )(quantized.reshape(b, hw, d_pad), conv_w[:, :, 0, 0], cp)

    return spikes4.reshape(t_steps, b, c, h, w), encoding_indices


# speedup vs baseline: 1.0218x; 1.0082x over previous
import functools

import jax
import jax.numpy as jnp
from jax import lax
from jax.experimental import pallas as pl
from jax.experimental.pallas import tpu as pltpu
from jax.experimental.pallas import tpu_sc as plsc

T_STEPS = 16
TAU_LIF = 2.0
V_TH = 1.0
BN_EPS = 1e-5


def _bf16_rtne(v):
    u = lax.bitcast_convert_type(v, jnp.int32)
    r = u + ((u >> 16) & 1) + 0x7FFF
    r = r & jnp.int32(-65536)
    return lax.bitcast_convert_type(r, jnp.float32).astype(jnp.bfloat16)


def _memout_body(coef_s, alpha_s, x_ref, xm_ref):
    a = alpha_s[0, 0]
    x0 = x_ref[0]
    s1 = x0 * coef_s[0, 0]
    s2 = x0
    for t in range(1, T_STEPS):
        xt = x_ref[t]
        s1 = s1 + xt * coef_s[0, t]
        s2 = s2 + xt
    xm_ref[...] = (1.0 - a) * s1 + (a * s2) * 0.0625


def _head_body(q_ref, w_ref, cp_ref, out_ref):
    c = w_ref.shape[1]
    w = w_ref[...]
    wb = _bf16_rtne(w)
    for i in range(q_ref.shape[0]):
        q = q_ref[i][:, :c]
        h = lax.dot_general(wb, _bf16_rtne(q),
                            (((1,), (1,)), ((), ())),
                            preferred_element_type=jnp.float32)
        h = h + cp_ref[:, 0:1]
        h = (h - cp_ref[:, 3:4]) / jnp.sqrt(cp_ref[:, 4:5] + BN_EPS)
        h = h * cp_ref[:, 1:2] + cp_ref[:, 2:3]
        v = jnp.zeros_like(h)
        for t in range(T_STEPS):
            v = v + (h - v) / TAU_LIF
            s = (v - V_TH >= 0.0).astype(jnp.float32)
            out_ref[t, i] = s
            v = (1.0 - s) * v


def _sc_gather(embeddings, indices):
    n = indices.shape[0]
    k_rows, d = embeddings.shape
    d_pad = 128
    emb_pad = jnp.pad(embeddings, ((0, 0), (0, d_pad - d)))
    info = plsc.get_sparse_core_info()
    nc, ns = info.num_cores, info.num_subcores
    nw = nc * ns
    b_per_w = n // nw
    mesh = plsc.VectorSubcoreMesh(core_axis_name="c", subcore_axis_name="s")

    @functools.partial(
        pl.kernel,
        out_type=jax.ShapeDtypeStruct((n, d_pad), jnp.float32),
        mesh=mesh,
        scratch_types=[
            pltpu.VMEM((b_per_w,), jnp.int32),
            pltpu.VMEM((b_per_w, d_pad), jnp.float32),
            pltpu.SemaphoreType.DMA,
        ],
    )
    def gather_kernel(emb_hbm, idx_hbm, out_hbm, idx_v, rows_v, sem):
        wid = lax.axis_index("s") * nc + lax.axis_index("c")
        base = wid * b_per_w
        pltpu.sync_copy(idx_hbm.at[pl.ds(base, b_per_w)], idx_v)
        pltpu.async_copy(emb_hbm.at[idx_v], rows_v, sem).wait()
        pltpu.sync_copy(rows_v, out_hbm.at[pl.ds(base, b_per_w)])

    return gather_kernel(emb_pad, indices)


def kernel(x, embeddings, alpha, conv_w, conv_b, bn_gamma, bn_beta,
           bn_mean, bn_var):
    t_steps, b, c, h, w = x.shape
    hw = h * w
    k_rows, d = embeddings.shape
    x4 = x.reshape(t_steps, b, c, hw)
    coef = jnp.power(0.8, jnp.arange(t_steps - 1, -1, -1,
                                     dtype=jnp.float32)).reshape(1, t_steps)
    alpha2 = jnp.asarray(alpha, jnp.float32).reshape(1, 1)

    bpair = 2
    xm3 = pl.pallas_call(
        _memout_body,
        grid=(b // bpair,),
        in_specs=[
            pl.BlockSpec(memory_space=pltpu.SMEM),
            pl.BlockSpec(memory_space=pltpu.SMEM),
            pl.BlockSpec((t_steps, bpair, c, hw), lambda bb: (0, bb, 0, 0)),
        ],
        out_specs=pl.BlockSpec((bpair, c, hw), lambda bb: (bb, 0, 0)),
        out_shape=jax.ShapeDtypeStruct((b, c, hw), jnp.float32),
    )(coef, alpha2, x4)

    flat_x = jnp.transpose(xm3.reshape(b, c, h, w), (0, 2, 3, 1)).reshape(-1, d)
    distances = (jnp.sum(flat_x ** 2, axis=1, keepdims=True)
                 + jnp.sum(embeddings ** 2, axis=1)
                 - 2.0 * jnp.matmul(flat_x, embeddings.T))
    encoding_indices = jnp.argmin(distances, axis=1)

    quantized = _sc_gather(embeddings, encoding_indices)
    d_pad = quantized.shape[1]

    cp = jnp.stack([conv_b, bn_gamma, bn_beta, bn_mean, bn_var],
                   axis=1).astype(jnp.float32)
    spikes4 = pl.pallas_call(
        _head_body,
        grid=(b // bpair,),
        in_specs=[
            pl.BlockSpec((bpair, hw, d_pad), lambda bb: (bb, 0, 0)),
            pl.BlockSpec((c, c), lambda bb: (0, 0)),
            pl.BlockSpec(cp.shape, lambda bb: (0, 0)),
        ],
        out_specs=pl.BlockSpec((t_steps, bpair, c, hw), lambda bb: (0, bb, 0, 0)),
        out_shape=jax.ShapeDtypeStruct((t_steps, b, c, hw), jnp.float32),
    )(quantized.reshape(b, hw, d_pad), conv_w[:, :, 0, 0], cp)

    return spikes4.reshape(t_steps, b, c, h, w), encoding_indices
